# Initial kernel scaffold; baseline (speedup 1.0000x reference)
import jax
import jax.numpy as jnp
from jax.experimental import pallas as pl


def kernel(coordinates, image):
    raise NotImplementedError("write your pallas kernel here")


# speedup vs baseline: 4.1563x; 4.1563x over previous
import functools

import jax
import jax.numpy as jnp
from jax import lax
from jax.experimental import pallas as pl
from jax.experimental.pallas import tpu as pltpu
from jax.experimental.pallas import tpu_sc as plsc

H = 512
W = 512
C = 3
N = 2097152

LANES = 16
NC = 2
NS = 16
NW = NC * NS

CPT = N // NW
CHUNK = 2048
NCHUNK = CPT // CHUNK
GSEG = 128
NSEG = CHUNK // GSEG
D = 8


def _lane_gather(v, idx):
    dnums = lax.GatherDimensionNumbers(
        offset_dims=(), collapsed_slice_dims=(0,), start_index_map=(0,)
    )
    return lax.gather(
        v, idx[:, None], dnums, (1,),
        mode=lax.GatherScatterMode.PROMISE_IN_BOUNDS,
    )


def _sc_lookup(coords_flat, table):
    mesh = plsc.VectorSubcoreMesh(core_axis_name="c", subcore_axis_name="s")

    @functools.partial(
        pl.kernel,
        mesh=mesh,
        out_type=jax.ShapeDtypeStruct((N, C), jnp.float32),
        compiler_params=pltpu.CompilerParams(use_tc_tiling_on_sc=False),
        scratch_types=[
            pltpu.VMEM((2 * CHUNK,), jnp.float32),
            pltpu.VMEM((NSEG, GSEG), jnp.int32),
            pltpu.VMEM((NSEG, GSEG, D), jnp.float32),
            pltpu.SemaphoreType.DMA,
        ],
    )
    def k(coords_hbm, table_hbm, out_hbm, cbuf, ibuf, rbuf, sem):
        wid = lax.axis_index("s") * NC + lax.axis_index("c")
        base = wid * CPT

        lanesv = lax.iota(jnp.int32, LANES)
        mult = jnp.where((lanesv & 1) == 0, W, 1)
        swap = lanesv ^ 1
        epat = (lanesv * 2) & (LANES - 1)
        lowhalf = lanesv < 8

        def chunk_body(g, carry):
            cstart = base + g * CHUNK
            pltpu.sync_copy(
                coords_hbm.at[pl.ds(2 * cstart, 2 * CHUNK)], cbuf
            )

            def seg_body(j, carry):
                for kk in range(GSEG // LANES):
                    o = 2 * (GSEG * j + LANES * kk)
                    a = cbuf[pl.ds(o, LANES)].astype(jnp.int32)
                    b = cbuf[pl.ds(o + LANES, LANES)].astype(jnp.int32)
                    a = jnp.minimum(jnp.maximum(a, 0), W - 1) * mult
                    b = jnp.minimum(jnp.maximum(b, 0), W - 1) * mult
                    sa = a + _lane_gather(a, swap)
                    sb = b + _lane_gather(b, swap)
                    pix = jnp.where(
                        lowhalf,
                        _lane_gather(sa, epat),
                        _lane_gather(sb, epat),
                    )
                    ibuf[j, pl.ds(LANES * kk, LANES)] = pix
                return carry

            lax.fori_loop(0, NSEG, seg_body, 0)

            copies = [
                pltpu.async_copy(table_hbm.at[ibuf.at[j]], rbuf.at[j], sem)
                for j in range(NSEG)
            ]
            for cp in copies:
                cp.wait()
            for j in range(NSEG):
                pltpu.sync_copy(
                    rbuf.at[j, :, pl.ds(0, C)],
                    out_hbm.at[pl.ds(cstart + GSEG * j, GSEG), :],
                )
            return carry

        lax.fori_loop(0, NCHUNK, chunk_body, 0)

    return k(coords_flat, table)


def kernel(coordinates, image):
    table = (image * jnp.float32(1.0 / 255.0)).reshape(H * W, C)
    table = jnp.pad(table, ((0, 0), (0, D - C)))
    coords_flat = coordinates.reshape(-1)
    return _sc_lookup(coords_flat, table)


# speedup vs baseline: 7.1139x; 1.7116x over previous
import functools

import jax
import jax.numpy as jnp
from jax import lax
from jax.experimental import pallas as pl
from jax.experimental.pallas import tpu as pltpu
from jax.experimental.pallas import tpu_sc as plsc

H = 512
W = 512
C = 3
N = 2097152

LANES = 16
NC = 2
NS = 16
NW = NC * NS

CPT = N // NW
CHUNK = 2048
NCHUNK = CPT // CHUNK
GSEG = 128
NSEGE = C * CHUNK // GSEG
NGRP = CHUNK // LANES


def _lane_gather(v, idx):
    dnums = lax.GatherDimensionNumbers(
        offset_dims=(), collapsed_slice_dims=(0,), start_index_map=(0,)
    )
    return lax.gather(
        v, idx[:, None], dnums, (1,),
        mode=lax.GatherScatterMode.PROMISE_IN_BOUNDS,
    )


def _sc_lookup(coords_flat, image_flat):
    mesh = plsc.VectorSubcoreMesh(core_axis_name="c", subcore_axis_name="s")

    @functools.partial(
        pl.kernel,
        mesh=mesh,
        out_type=jax.ShapeDtypeStruct((N * C,), jnp.float32),
        compiler_params=pltpu.CompilerParams(use_tc_tiling_on_sc=False),
        scratch_types=[
            pltpu.VMEM((2 * CHUNK,), jnp.float32),
            pltpu.VMEM((C * CHUNK,), jnp.int32),
            pltpu.VMEM((C * CHUNK,), jnp.float32),
            pltpu.SemaphoreType.DMA,
        ],
    )
    def k(coords_hbm, img_hbm, out_hbm, cbuf, ebuf, obuf, sem):
        wid = lax.axis_index("s") * NC + lax.axis_index("c")
        base = wid * CPT

        lanesv = lax.iota(jnp.int32, LANES)
        mult = jnp.where((lanesv & 1) == 0, W, 1)
        swap = lanesv ^ 1
        epat = (lanesv * 2) & (LANES - 1)
        lowhalf = lanesv < 8
        inv255 = jnp.float32(1.0 / 255.0)

        cpat = []
        rpat = []
        for t in range(C):
            e = lanesv + (LANES * t)
            q = (e * 21846) >> 16
            cpat.append(q)
            rpat.append(e - C * q)

        def chunk_body(g, carry):
            cstart = base + g * CHUNK
            pltpu.sync_copy(
                coords_hbm.at[pl.ds(2 * cstart, 2 * CHUNK)], cbuf
            )

            def grp_body(i, carry):
                for kk in range(4):
                    gi = 4 * i + kk
                    o = 2 * LANES * gi
                    a = cbuf[pl.ds(o, LANES)].astype(jnp.int32)
                    b = cbuf[pl.ds(o + LANES, LANES)].astype(jnp.int32)
                    a = jnp.minimum(jnp.maximum(a, 0), W - 1) * mult
                    b = jnp.minimum(jnp.maximum(b, 0), W - 1) * mult
                    sa = a + _lane_gather(a, swap)
                    sb = b + _lane_gather(b, swap)
                    pix = jnp.where(
                        lowhalf,
                        _lane_gather(sa, epat),
                        _lane_gather(sb, epat),
                    )
                    for t in range(C):
                        ev = C * _lane_gather(pix, cpat[t]) + rpat[t]
                        ebuf[pl.ds(C * LANES * gi + LANES * t, LANES)] = ev
                return carry

            lax.fori_loop(0, NGRP // 4, grp_body, 0)

            copies = [
                pltpu.async_copy(
                    img_hbm.at[ebuf.at[pl.ds(GSEG * j, GSEG)]],
                    obuf.at[pl.ds(GSEG * j, GSEG)],
                    sem,
                )
                for j in range(NSEGE)
            ]
            for cp in copies:
                cp.wait()

            def scale_body(i, carry):
                for kk in range(8):
                    o = LANES * (8 * i + kk)
                    obuf[pl.ds(o, LANES)] = obuf[pl.ds(o, LANES)] * inv255
                return carry

            lax.fori_loop(0, C * CHUNK // (8 * LANES), scale_body, 0)

            pltpu.sync_copy(
                obuf, out_hbm.at[pl.ds(C * cstart, C * CHUNK)]
            )
            return carry

        lax.fori_loop(0, NCHUNK, chunk_body, 0)

    return k(coords_flat, image_flat)


def kernel(coordinates, image):
    out = _sc_lookup(coordinates.reshape(-1), image.reshape(-1))
    return out.reshape(N, C)
